# Initial kernel scaffold; baseline (speedup 1.0000x reference)
import jax
import jax.numpy as jnp
from jax.experimental import pallas as pl


def kernel(logits, imp_tau, exc_kappa, imp_i, imp_j, exc_i, exc_j):
    raise NotImplementedError("write your pallas kernel here")


# speedup vs baseline: 12.8009x; 12.8009x over previous
import jax
import jax.numpy as jnp
from jax import lax
from jax.experimental import pallas as pl
from jax.experimental.pallas import tpu as pltpu

_ROWS = 512


def _proj_body(gidx_ref, sidx_ref, aux_ref, x_ref, o_ref):
    x = x_ref[...]
    p = jax.nn.sigmoid(x)
    cols = x.shape[1]

    grow = gidx_ref[0:1, :]
    gsel = (lax.broadcasted_iota(jnp.int32, (cols, 128), 0) == grow).astype(
        jnp.float32
    )
    q = lax.dot_general(
        p, gsel, (((1,), (0,)), ((), ())), preferred_element_type=jnp.float32
    )
    qi_imp = q[:, 0:32]
    qj_imp = q[:, 32:64]
    qi_exc = q[:, 64:96]
    qj_exc = q[:, 96:128]

    tau = aux_ref[0:1, 0:32]
    kap = aux_ref[1:2, 0:32]

    qj_imp_n = jnp.minimum(jnp.maximum(qj_imp, qi_imp + tau), 1.0)

    s = qi_exc + qj_exc
    dd = qi_exc - qj_exc
    viol = s > kap
    cap = jnp.minimum(kap, 1.0)
    qi_exc_n = jnp.where(viol, jnp.clip((kap + dd) * 0.5, 0.0, cap), qi_exc)
    qj_exc_n = jnp.where(viol, jnp.clip((kap - dd) * 0.5, 0.0, cap), qj_exc)

    upd = jnp.concatenate([qj_imp_n, qi_exc_n, qj_exc_n], axis=1)

    scol = sidx_ref[:, 0:1]
    ssel = (lax.broadcasted_iota(jnp.int32, (96, cols), 1) == scol).astype(
        jnp.float32
    )
    mask = jnp.sum(ssel, axis=0, keepdims=True)
    scat = lax.dot_general(
        upd, ssel, (((1,), (0,)), ((), ())), preferred_element_type=jnp.float32
    )
    o_ref[...] = p * (1.0 - mask) + scat


def kernel(logits, imp_tau, exc_kappa, imp_i, imp_j, exc_i, exc_j):
    rows, cols = logits.shape
    gidx = jnp.concatenate(
        [imp_i, imp_j, exc_i, exc_j]
    ).astype(jnp.int32)
    gidx2 = jnp.broadcast_to(gidx[None, :], (8, 128))
    sidx = gidx[32:]
    sidx2 = jnp.broadcast_to(sidx[:, None], (96, 128))
    aux = jnp.zeros((8, 128), jnp.float32)
    aux = aux.at[0, :32].set(imp_tau.astype(jnp.float32))
    aux = aux.at[1, :32].set(exc_kappa.astype(jnp.float32))

    grid = rows // _ROWS
    return pl.pallas_call(
        _proj_body,
        grid=(grid,),
        in_specs=[
            pl.BlockSpec((8, 128), lambda i: (0, 0)),
            pl.BlockSpec((96, 128), lambda i: (0, 0)),
            pl.BlockSpec((8, 128), lambda i: (0, 0)),
            pl.BlockSpec((_ROWS, cols), lambda i: (i, 0)),
        ],
        out_specs=pl.BlockSpec((_ROWS, cols), lambda i: (i, 0)),
        out_shape=jax.ShapeDtypeStruct((rows, cols), jnp.float32),
    )(gidx2, sidx2, aux, logits)
